# baseline (device time: 338627 ns/iter reference)
import jax
import jax.numpy as jnp
from jax import lax
from jax.experimental import pallas as pl
from jax.experimental.pallas import tpu as pltpu

N_DEV = 4
SQ = 2048
H = 8
D = 128
DM = 1024
HD = H * D
QT = 256
N_T = SQ // QT
CH = 4
CR = SQ // CH
SCALE = 0.08838834764831843


def kernel(x, Wq, K_ext, V_ext, Wo):
    xb = x.reshape(SQ, DM).astype(jnp.bfloat16)
    wqb = Wq.astype(jnp.bfloat16)
    wob = Wo.astype(jnp.bfloat16)
    kb = K_ext.reshape(SQ, N_DEV, HD).transpose(1, 0, 2).astype(jnp.bfloat16)
    vb = V_ext.reshape(SQ, N_DEV, HD).transpose(1, 0, 2).astype(jnp.bfloat16)

    def body(x_ref, wq_ref, wo_ref, k_hbm, v_hbm, out_ref,
             kf, vf, q_s, ctx_s, part, slots, rbuf,
             kv_send, kv_recv, rrecv, rsend, loc_sem,
             ar_sR, ar_sL, ar_recv, fwd_a, fwd_b):
        my = lax.axis_index("i")
        right = (my + 1) % N_DEV
        left = (my + 3) % N_DEV
        MESH = pl.DeviceIdType.MESH

        bar = pltpu.get_barrier_semaphore()
        for o in (1, 2, 3):
            pl.semaphore_signal(bar, inc=1, device_id=((my + o) % N_DEV,),
                                device_id_type=MESH)
        pl.semaphore_wait(bar, 3)

        si = 0
        scat = []
        for c in range(CH):
            scat.append(pltpu.make_async_remote_copy(
                src_ref=k_hbm.at[2, c * CR:(c + 1) * CR, :],
                dst_ref=rbuf.at[c], send_sem=kv_send.at[si],
                recv_sem=rrecv.at[c], device_id=(1,), device_id_type=MESH))
            si += 1
        for c in range(CH):
            scat.append(pltpu.make_async_remote_copy(
                src_ref=v_hbm.at[2, c * CR:(c + 1) * CR, :],
                dst_ref=rbuf.at[c], send_sem=kv_send.at[si],
                recv_sem=rrecv.at[c], device_id=(3,), device_id_type=MESH))
            si += 1
        for c in range(CH):
            for p, src, dstr, rsl in ((1, k_hbm, kf, 0), (1, v_hbm, vf, 1),
                                      (3, k_hbm, kf, 0), (3, v_hbm, vf, 1)):
                scat.append(pltpu.make_async_remote_copy(
                    src_ref=src.at[p, c * CR:(c + 1) * CR, :],
                    dst_ref=dstr.at[c * CR:(c + 1) * CR, :],
                    send_sem=kv_send.at[si], recv_sem=kv_recv.at[c, rsl],
                    device_id=(p,), device_id_type=MESH))
                si += 1
        ck = [pltpu.make_async_copy(k_hbm.at[0], kf, loc_sem.at[0])]
        cv = [pltpu.make_async_copy(v_hbm.at[0], vf, loc_sem.at[1])]
        rwait = [pltpu.make_async_remote_copy(
            src_ref=rbuf.at[c], dst_ref=rbuf.at[c], send_sem=rsend.at[c],
            recv_sem=rrecv.at[c], device_id=(0,), device_id_type=MESH)
            for c in range(CH)]
        rfwd_k = [pltpu.make_async_remote_copy(
            src_ref=rbuf.at[c], dst_ref=kf.at[c * CR:(c + 1) * CR, :],
            send_sem=rsend.at[c], recv_sem=kv_recv.at[c, 0],
            device_id=(2,), device_id_type=MESH) for c in range(CH)]
        rfwd_v = [pltpu.make_async_remote_copy(
            src_ref=rbuf.at[c], dst_ref=vf.at[c * CR:(c + 1) * CR, :],
            send_sem=rsend.at[c], recv_sem=kv_recv.at[c, 1],
            device_id=(2,), device_id_type=MESH) for c in range(CH)]
        wk = [pltpu.make_async_remote_copy(
            src_ref=kf.at[c * CR:(c + 1) * CR, :],
            dst_ref=kf.at[c * CR:(c + 1) * CR, :], send_sem=rsend.at[c],
            recv_sem=kv_recv.at[c, 0], device_id=(0,), device_id_type=MESH)
            for c in range(CH)]
        wv = [pltpu.make_async_remote_copy(
            src_ref=vf.at[c * CR:(c + 1) * CR, :],
            dst_ref=vf.at[c * CR:(c + 1) * CR, :], send_sem=rsend.at[c],
            recv_sem=kv_recv.at[c, 1], device_id=(0,), device_id_type=MESH)
            for c in range(CH)]

        pR = [pltpu.make_async_remote_copy(
            src_ref=part.at[t], dst_ref=slots.at[2, t],
            send_sem=ar_sR.at[t], recv_sem=ar_recv.at[t, 2],
            device_id=(right,), device_id_type=MESH) for t in range(N_T)]
        pL = [pltpu.make_async_remote_copy(
            src_ref=part.at[t], dst_ref=slots.at[0, t],
            send_sem=ar_sL.at[t], recv_sem=ar_recv.at[t, 0],
            device_id=(left,), device_id_type=MESH) for t in range(N_T)]
        wslot = [[pltpu.make_async_remote_copy(
            src_ref=part.at[t], dst_ref=slots.at[s, t],
            send_sem=ar_sR.at[t], recv_sem=ar_recv.at[t, s],
            device_id=(0,), device_id_type=MESH) for s in range(3)]
            for t in range(N_T)]
        fA = [pltpu.make_async_remote_copy(
            src_ref=slots.at[2, t], dst_ref=slots.at[1, t],
            send_sem=fwd_a.at[t], recv_sem=ar_recv.at[t, 1],
            device_id=(right,), device_id_type=MESH) for t in range(N_T)]
        fB = [pltpu.make_async_remote_copy(
            src_ref=slots.at[0, t], dst_ref=slots.at[1, t],
            send_sem=fwd_b.at[t], recv_sem=ar_recv.at[t, 1],
            device_id=(1,), device_id_type=MESH) for t in range(N_T)]

        @pl.when(my == 0)
        def _():
            for r in ck + cv:
                r.start()
            for r in scat:
                r.start()

        for t in range(N_T):
            if t % 2 == 0:
                c = t // 2

                if c == 0:
                    @pl.when(my == 0)
                    def _():
                        for r in ck + cv:
                            r.wait()

                @pl.when(my == 1)
                def _(c=c):
                    rwait[c].wait_recv()
                    rfwd_k[c].start()

                @pl.when(my == 3)
                def _(c=c):
                    rwait[c].wait_recv()
                    rfwd_v[c].start()

                @pl.when(my != 0)
                def _(c=c):
                    wk[c].wait_recv()
                    wv[c].wait_recv()

            kv_len = QT * (t + 1)
            xt = x_ref[pl.ds(t * QT, QT), :]
            q_s[:, :] = (lax.dot_general(xt, wq_ref[:, :],
                                         (((1,), (0,)), ((), ())),
                                         preferred_element_type=jnp.float32)
                         * SCALE).astype(jnp.bfloat16)

            rows = t * QT + lax.broadcasted_iota(jnp.int32, (QT, kv_len), 0)
            cols = lax.broadcasted_iota(jnp.int32, (QT, kv_len), 1)
            bias = jnp.where((cols // 64) <= (rows // 64),
                             jnp.float32(0.0), jnp.float32(-1e9))

            def head_body(h, _, t=t, kv_len=kv_len, bias=bias):
                hoff = pl.multiple_of(h * D, D)
                qh = q_s[:, pl.ds(hoff, D)]
                kh = kf[pl.ds(0, kv_len), pl.ds(hoff, D)]
                s = lax.dot_general(qh, kh, (((1,), (1,)), ((), ())),
                                    preferred_element_type=jnp.float32)
                s = s + bias
                m = jnp.max(s, axis=1, keepdims=True)
                w = jnp.exp(s - m)
                r = 1.0 / jnp.sum(w, axis=1, keepdims=True)
                w = (w * r).astype(jnp.bfloat16)
                vh = vf[pl.ds(0, kv_len), pl.ds(hoff, D)]
                ctxh = lax.dot_general(w, vh, (((1,), (0,)), ((), ())),
                                       preferred_element_type=jnp.float32)
                ctx_s[:, pl.ds(hoff, D)] = ctxh.astype(jnp.bfloat16)
                return 0

            lax.fori_loop(0, H, head_body, 0)
            out_t = lax.dot_general(ctx_s[:, :], wo_ref[:, :],
                                    (((1,), (0,)), ((), ())),
                                    preferred_element_type=jnp.float32)
            out_ref[pl.ds(t * QT, QT), :] = out_t
            part[t, :, :] = out_t.astype(jnp.bfloat16)

            @pl.when(my != 0)
            def _(t=t):
                pR[t].start()
                pL[t].start()

        @pl.when(my == 0)
        def _():
            for r in scat:
                r.wait_send()
            for t in range(N_T):
                pR[t].start()
                pL[t].start()

        for t in range(N_T):
            wslot[t][2].wait_recv()

            @pl.when(my != 0)
            def _(t=t):
                fA[t].start()

            wslot[t][0].wait_recv()

            @pl.when(my == 2)
            def _(t=t):
                fB[t].start()

            wslot[t][1].wait_recv()
            rs = pl.ds(t * QT, QT)
            out_ref[rs, :] = (out_ref[rs, :]
                              + slots[0, t].astype(jnp.float32)
                              + slots[1, t].astype(jnp.float32)
                              + slots[2, t].astype(jnp.float32))

        @pl.when(my == 1)
        def _():
            for c in range(CH):
                rfwd_k[c].wait_send()

        @pl.when(my == 3)
        def _():
            for c in range(CH):
                rfwd_v[c].wait_send()

        for t in range(N_T):
            pR[t].wait_send()
            pL[t].wait_send()

        @pl.when(my != 0)
        def _():
            for t in range(N_T):
                fA[t].wait_send()

        @pl.when(my == 2)
        def _():
            for t in range(N_T):
                fB[t].wait_send()

    out = pl.pallas_call(
        body,
        out_shape=jax.ShapeDtypeStruct((SQ, HD), jnp.float32),
        in_specs=[
            pl.BlockSpec(memory_space=pltpu.VMEM),
            pl.BlockSpec(memory_space=pltpu.VMEM),
            pl.BlockSpec(memory_space=pltpu.VMEM),
            pl.BlockSpec(memory_space=pl.ANY),
            pl.BlockSpec(memory_space=pl.ANY),
        ],
        out_specs=pl.BlockSpec(memory_space=pltpu.VMEM),
        scratch_shapes=[
            pltpu.VMEM((SQ, HD), jnp.bfloat16),
            pltpu.VMEM((SQ, HD), jnp.bfloat16),
            pltpu.VMEM((QT, HD), jnp.bfloat16),
            pltpu.VMEM((QT, HD), jnp.bfloat16),
            pltpu.VMEM((N_T, QT, HD), jnp.bfloat16),
            pltpu.VMEM((3, N_T, QT, HD), jnp.bfloat16),
            pltpu.VMEM((CH, CR, HD), jnp.bfloat16),
            pltpu.SemaphoreType.DMA((24,)),
            pltpu.SemaphoreType.DMA((CH, 2)),
            pltpu.SemaphoreType.DMA((CH,)),
            pltpu.SemaphoreType.DMA((CH,)),
            pltpu.SemaphoreType.DMA((2,)),
            pltpu.SemaphoreType.DMA((N_T,)),
            pltpu.SemaphoreType.DMA((N_T,)),
            pltpu.SemaphoreType.DMA((N_T, 3)),
            pltpu.SemaphoreType.DMA((N_T,)),
            pltpu.SemaphoreType.DMA((N_T,)),
        ],
        compiler_params=pltpu.CompilerParams(collective_id=0),
    )(xb, wqb, wob, kb, vb)
    return out.reshape(1, SQ, HD)


# device time: 319531 ns/iter; 1.0598x vs baseline; 1.0598x over previous
import jax
import jax.numpy as jnp
from jax import lax
from jax.experimental import pallas as pl
from jax.experimental.pallas import tpu as pltpu

N_DEV = 4
SQ = 2048
H = 8
D = 128
DM = 1024
HD = H * D
QT = 256
N_T = SQ // QT
CH = 4
CR = SQ // CH
SCALE = 0.08838834764831843


def kernel(x, Wq, K_ext, V_ext, Wo):
    xb = x.reshape(SQ, DM).astype(jnp.bfloat16)
    wqb = Wq.astype(jnp.bfloat16)
    wob = Wo.astype(jnp.bfloat16)
    kb = K_ext.reshape(SQ, N_DEV * HD).astype(jnp.bfloat16)
    vb = V_ext.reshape(SQ, N_DEV * HD).astype(jnp.bfloat16)

    def body(x_ref, wq_ref, wo_ref, k_hbm, v_hbm, out_ref,
             kf, vf, q_s, ctx_s, part, slots, rbuf,
             kv_send, kv_recv, rrecv, rsend, loc_sem,
             ar_sR, ar_sL, ar_recv, fwd_a, fwd_b):
        my = lax.axis_index("i")
        right = (my + 1) % N_DEV
        left = (my + 3) % N_DEV
        MESH = pl.DeviceIdType.MESH

        bar = pltpu.get_barrier_semaphore()
        for o in (1, 2, 3):
            pl.semaphore_signal(bar, inc=1, device_id=((my + o) % N_DEV,),
                                device_id_type=MESH)
        pl.semaphore_wait(bar, 3)

        si = 0
        scat = []
        for c in range(CH):
            scat.append(pltpu.make_async_remote_copy(
                src_ref=k_hbm.at[c * CR:(c + 1) * CR, 2 * HD:3 * HD],
                dst_ref=rbuf.at[c], send_sem=kv_send.at[si],
                recv_sem=rrecv.at[c], device_id=(1,), device_id_type=MESH))
            si += 1
        for c in range(CH):
            scat.append(pltpu.make_async_remote_copy(
                src_ref=v_hbm.at[c * CR:(c + 1) * CR, 2 * HD:3 * HD],
                dst_ref=rbuf.at[c], send_sem=kv_send.at[si],
                recv_sem=rrecv.at[c], device_id=(3,), device_id_type=MESH))
            si += 1
        for c in range(CH):
            for p, src, dstr, rsl in ((1, k_hbm, kf, 0), (1, v_hbm, vf, 1),
                                      (3, k_hbm, kf, 0), (3, v_hbm, vf, 1)):
                scat.append(pltpu.make_async_remote_copy(
                    src_ref=src.at[c * CR:(c + 1) * CR, p * HD:(p + 1) * HD],
                    dst_ref=dstr.at[c * CR:(c + 1) * CR, :],
                    send_sem=kv_send.at[si], recv_sem=kv_recv.at[c, rsl],
                    device_id=(p,), device_id_type=MESH))
                si += 1
        ck = [pltpu.make_async_copy(k_hbm.at[:, 0:HD], kf, loc_sem.at[0])]
        cv = [pltpu.make_async_copy(v_hbm.at[:, 0:HD], vf, loc_sem.at[1])]
        rwait = [pltpu.make_async_remote_copy(
            src_ref=rbuf.at[c], dst_ref=rbuf.at[c], send_sem=rsend.at[c],
            recv_sem=rrecv.at[c], device_id=(0,), device_id_type=MESH)
            for c in range(CH)]
        rfwd_k = [pltpu.make_async_remote_copy(
            src_ref=rbuf.at[c], dst_ref=kf.at[c * CR:(c + 1) * CR, :],
            send_sem=rsend.at[c], recv_sem=kv_recv.at[c, 0],
            device_id=(2,), device_id_type=MESH) for c in range(CH)]
        rfwd_v = [pltpu.make_async_remote_copy(
            src_ref=rbuf.at[c], dst_ref=vf.at[c * CR:(c + 1) * CR, :],
            send_sem=rsend.at[c], recv_sem=kv_recv.at[c, 1],
            device_id=(2,), device_id_type=MESH) for c in range(CH)]
        wk = [pltpu.make_async_remote_copy(
            src_ref=kf.at[c * CR:(c + 1) * CR, :],
            dst_ref=kf.at[c * CR:(c + 1) * CR, :], send_sem=rsend.at[c],
            recv_sem=kv_recv.at[c, 0], device_id=(0,), device_id_type=MESH)
            for c in range(CH)]
        wv = [pltpu.make_async_remote_copy(
            src_ref=vf.at[c * CR:(c + 1) * CR, :],
            dst_ref=vf.at[c * CR:(c + 1) * CR, :], send_sem=rsend.at[c],
            recv_sem=kv_recv.at[c, 1], device_id=(0,), device_id_type=MESH)
            for c in range(CH)]

        pR = [pltpu.make_async_remote_copy(
            src_ref=part.at[t], dst_ref=slots.at[2, t],
            send_sem=ar_sR.at[t], recv_sem=ar_recv.at[t, 2],
            device_id=(right,), device_id_type=MESH) for t in range(N_T)]
        pL = [pltpu.make_async_remote_copy(
            src_ref=part.at[t], dst_ref=slots.at[0, t],
            send_sem=ar_sL.at[t], recv_sem=ar_recv.at[t, 0],
            device_id=(left,), device_id_type=MESH) for t in range(N_T)]
        wslot = [[pltpu.make_async_remote_copy(
            src_ref=part.at[t], dst_ref=slots.at[s, t],
            send_sem=ar_sR.at[t], recv_sem=ar_recv.at[t, s],
            device_id=(0,), device_id_type=MESH) for s in range(3)]
            for t in range(N_T)]
        fA = [pltpu.make_async_remote_copy(
            src_ref=slots.at[2, t], dst_ref=slots.at[1, t],
            send_sem=fwd_a.at[t], recv_sem=ar_recv.at[t, 1],
            device_id=(right,), device_id_type=MESH) for t in range(N_T)]
        fB = [pltpu.make_async_remote_copy(
            src_ref=slots.at[0, t], dst_ref=slots.at[1, t],
            send_sem=fwd_b.at[t], recv_sem=ar_recv.at[t, 1],
            device_id=(1,), device_id_type=MESH) for t in range(N_T)]

        @pl.when(my == 0)
        def _():
            for r in ck + cv:
                r.start()
            for r in scat:
                r.start()

        for t in range(N_T):
            if t % 2 == 0:
                c = t // 2

                if c == 0:
                    @pl.when(my == 0)
                    def _():
                        for r in ck + cv:
                            r.wait()

                @pl.when(my == 1)
                def _(c=c):
                    rwait[c].wait_recv()
                    rfwd_k[c].start()

                @pl.when(my == 3)
                def _(c=c):
                    rwait[c].wait_recv()
                    rfwd_v[c].start()

                @pl.when(my != 0)
                def _(c=c):
                    wk[c].wait_recv()
                    wv[c].wait_recv()

            kv_len = QT * (t + 1)
            xt = x_ref[pl.ds(t * QT, QT), :]
            q_s[:, :] = (lax.dot_general(xt, wq_ref[:, :],
                                         (((1,), (0,)), ((), ())),
                                         preferred_element_type=jnp.float32)
                         * SCALE).astype(jnp.bfloat16)

            rows = t * QT + lax.broadcasted_iota(jnp.int32, (QT, kv_len), 0)
            cols = lax.broadcasted_iota(jnp.int32, (QT, kv_len), 1)
            bias = jnp.where((cols // 64) <= (rows // 64),
                             jnp.float32(0.0), jnp.float32(-1e9))

            def head_body(h, _, t=t, kv_len=kv_len, bias=bias):
                hoff = pl.multiple_of(h * D, D)
                qh = q_s[:, pl.ds(hoff, D)]
                kh = kf[pl.ds(0, kv_len), pl.ds(hoff, D)]
                s = lax.dot_general(qh, kh, (((1,), (1,)), ((), ())),
                                    preferred_element_type=jnp.float32)
                s = s + bias
                m = jnp.max(s, axis=1, keepdims=True)
                w = jnp.exp(s - m)
                r = 1.0 / jnp.sum(w, axis=1, keepdims=True)
                w = (w * r).astype(jnp.bfloat16)
                vh = vf[pl.ds(0, kv_len), pl.ds(hoff, D)]
                ctxh = lax.dot_general(w, vh, (((1,), (0,)), ((), ())),
                                       preferred_element_type=jnp.float32)
                ctx_s[:, pl.ds(hoff, D)] = ctxh.astype(jnp.bfloat16)
                return 0

            lax.fori_loop(0, H, head_body, 0)
            out_t = lax.dot_general(ctx_s[:, :], wo_ref[:, :],
                                    (((1,), (0,)), ((), ())),
                                    preferred_element_type=jnp.float32)
            out_ref[pl.ds(t * QT, QT), :] = out_t
            part[t, :, :] = out_t.astype(jnp.bfloat16)

            @pl.when(my != 0)
            def _(t=t):
                pR[t].start()
                pL[t].start()

        @pl.when(my == 0)
        def _():
            for r in scat:
                r.wait_send()
            for t in range(N_T):
                pR[t].start()
                pL[t].start()

        for t in range(N_T):
            wslot[t][2].wait_recv()

            @pl.when(my != 0)
            def _(t=t):
                fA[t].start()

            wslot[t][0].wait_recv()

            @pl.when(my == 2)
            def _(t=t):
                fB[t].start()

            wslot[t][1].wait_recv()
            rs = pl.ds(t * QT, QT)
            out_ref[rs, :] = (out_ref[rs, :]
                              + slots[0, t].astype(jnp.float32)
                              + slots[1, t].astype(jnp.float32)
                              + slots[2, t].astype(jnp.float32))

        @pl.when(my == 1)
        def _():
            for c in range(CH):
                rfwd_k[c].wait_send()

        @pl.when(my == 3)
        def _():
            for c in range(CH):
                rfwd_v[c].wait_send()

        for t in range(N_T):
            pR[t].wait_send()
            pL[t].wait_send()

        @pl.when(my != 0)
        def _():
            for t in range(N_T):
                fA[t].wait_send()

        @pl.when(my == 2)
        def _():
            for t in range(N_T):
                fB[t].wait_send()

    out = pl.pallas_call(
        body,
        out_shape=jax.ShapeDtypeStruct((SQ, HD), jnp.float32),
        in_specs=[
            pl.BlockSpec(memory_space=pltpu.VMEM),
            pl.BlockSpec(memory_space=pltpu.VMEM),
            pl.BlockSpec(memory_space=pltpu.VMEM),
            pl.BlockSpec(memory_space=pl.ANY),
            pl.BlockSpec(memory_space=pl.ANY),
        ],
        out_specs=pl.BlockSpec(memory_space=pltpu.VMEM),
        scratch_shapes=[
            pltpu.VMEM((SQ, HD), jnp.bfloat16),
            pltpu.VMEM((SQ, HD), jnp.bfloat16),
            pltpu.VMEM((QT, HD), jnp.bfloat16),
            pltpu.VMEM((QT, HD), jnp.bfloat16),
            pltpu.VMEM((N_T, QT, HD), jnp.bfloat16),
            pltpu.VMEM((3, N_T, QT, HD), jnp.bfloat16),
            pltpu.VMEM((CH, CR, HD), jnp.bfloat16),
            pltpu.SemaphoreType.DMA((24,)),
            pltpu.SemaphoreType.DMA((CH, 2)),
            pltpu.SemaphoreType.DMA((CH,)),
            pltpu.SemaphoreType.DMA((CH,)),
            pltpu.SemaphoreType.DMA((2,)),
            pltpu.SemaphoreType.DMA((N_T,)),
            pltpu.SemaphoreType.DMA((N_T,)),
            pltpu.SemaphoreType.DMA((N_T, 3)),
            pltpu.SemaphoreType.DMA((N_T,)),
            pltpu.SemaphoreType.DMA((N_T,)),
        ],
        compiler_params=pltpu.CompilerParams(collective_id=0),
    )(xb, wqb, wob, kb, vb)
    return out.reshape(1, SQ, HD)
